# Initial kernel scaffold; baseline (speedup 1.0000x reference)
import jax
import jax.numpy as jnp
from jax.experimental import pallas as pl


def kernel(input_ids, W_r, W_i, W_j, W_k):
    raise NotImplementedError("write your pallas kernel here")


# speedup vs baseline: 1.9309x; 1.9309x over previous
import functools

import jax
import jax.numpy as jnp
from jax import lax
from jax.experimental import pallas as pl
from jax.experimental.pallas import tpu as pltpu
from jax.experimental.pallas import tpu_sc as plsc

D = 64
NCOMP = 4
ROW = NCOMP * D
LANES = 16

NC = 2
NS = 16
NW = NC * NS

CHUNK = 128


@functools.partial(jax.jit, static_argnums=())
def _sc_embed(idx_flat, W_r, W_i, W_j, W_k):
    n = idx_flat.shape[0]
    per_w = n // NW
    n_chunks = per_w // CHUNK

    mesh = plsc.VectorSubcoreMesh(core_axis_name="c", subcore_axis_name="s")

    @functools.partial(
        pl.kernel,
        mesh=mesh,
        compiler_params=pltpu.CompilerParams(
            needs_layout_passes=False, use_tc_tiling_on_sc=False),
        out_type=jax.ShapeDtypeStruct((n, ROW), jnp.float32),
        scratch_types=[
            pltpu.VMEM((CHUNK,), jnp.int32),
            pltpu.VMEM((NCOMP, CHUNK, D), jnp.float32),
            pltpu.VMEM((CHUNK, ROW), jnp.float32),
            pltpu.SemaphoreType.DMA,
        ],
    )
    def kern(idx_hbm, wr_hbm, wi_hbm, wj_hbm, wk_hbm, out_hbm,
             idx_v, gbuf, obuf, sem):
        wid = lax.axis_index("s") * NC + lax.axis_index("c")
        base = wid * per_w

        lane = lax.broadcasted_iota(jnp.int32, (LANES,), 0)
        c_idx = lane & 3
        d_idx = [(lane >> 2) + (q * 4) for q in range(ROW // LANES)]

        @pl.loop(0, n_chunks)
        def _(g):
            start = base + g * CHUNK
            pltpu.sync_copy(idx_hbm.at[pl.ds(start, CHUNK)], idx_v)
            cps = [
                pltpu.async_copy(w.at[idx_v], gbuf.at[comp], sem)
                for comp, w in enumerate((wr_hbm, wi_hbm, wj_hbm, wk_hbm))
            ]
            for cp in cps:
                cp.wait()

            @pl.loop(0, CHUNK)
            def _(t):
                t_idx = jnp.full((LANES,), t, jnp.int32)
                for q in range(ROW // LANES):
                    vec = plsc.load_gather(gbuf, [c_idx, t_idx, d_idx[q]])
                    obuf[t, pl.ds(q * LANES, LANES)] = vec

            pltpu.sync_copy(obuf, out_hbm.at[pl.ds(start, CHUNK)])

    return kern(idx_flat, W_r, W_i, W_j, W_k)


def kernel(input_ids, W_r, W_i, W_j, W_k):
    batch, hist = input_ids.shape
    idx_flat = input_ids.reshape(batch * hist)
    out = _sc_embed(idx_flat, W_r, W_i, W_j, W_k)
    return out.reshape(batch, hist, D, NCOMP)


# speedup vs baseline: 2.0330x; 1.0529x over previous
import functools

import jax
import jax.numpy as jnp
from jax import lax
from jax.experimental import pallas as pl
from jax.experimental.pallas import tpu as pltpu
from jax.experimental.pallas import tpu_sc as plsc

D = 64
NCOMP = 4
ROW = NCOMP * D
LANES = 16

NC = 2
NS = 16
NW = NC * NS

CHUNK = 128


@jax.jit
def _sc_embed(idx_flat, W_r, W_i, W_j, W_k):
    n = idx_flat.shape[0]
    per_w = n // NW
    n_chunks = per_w // CHUNK
    assert n_chunks % 2 == 0

    mesh = plsc.VectorSubcoreMesh(core_axis_name="c", subcore_axis_name="s")

    @functools.partial(
        pl.kernel,
        mesh=mesh,
        compiler_params=pltpu.CompilerParams(
            needs_layout_passes=False, use_tc_tiling_on_sc=False),
        out_type=jax.ShapeDtypeStruct((n, ROW), jnp.float32),
        scratch_types=[
            pltpu.VMEM((2, CHUNK), jnp.int32),
            pltpu.VMEM((2, NCOMP, CHUNK, D), jnp.float32),
            pltpu.VMEM((CHUNK, ROW), jnp.float32),
            pltpu.SemaphoreType.DMA,
            pltpu.SemaphoreType.DMA,
            pltpu.SemaphoreType.DMA,
            pltpu.SemaphoreType.DMA,
            pltpu.SemaphoreType.DMA,
        ],
    )
    def kern(idx_hbm, wr_hbm, wi_hbm, wj_hbm, wk_hbm, out_hbm,
             idx_v, gbuf, obuf, gsem0, gsem1, isem0, isem1, osem):
        tables = (wr_hbm, wi_hbm, wj_hbm, wk_hbm)
        gsem = (gsem0, gsem1)
        isem = (isem0, isem1)

        wid = lax.axis_index("s") * NC + lax.axis_index("c")
        base = wid * per_w

        lane = lax.broadcasted_iota(jnp.int32, (LANES,), 0)
        c_idx = lane & 3
        d_idx = [(lane >> 2) + (q * 4) for q in range(ROW // LANES)]

        def idx_slice(g):
            return idx_hbm.at[pl.ds(base + g * CHUNK, CHUNK)]

        def out_slice(g):
            return out_hbm.at[pl.ds(base + g * CHUNK, CHUNK)]

        def fire_gathers(p):
            for comp, w in enumerate(tables):
                pltpu.async_copy(w.at[idx_v.at[p]], gbuf.at[p, comp], gsem[p])

        def wait_gathers(p):
            for comp, w in enumerate(tables):
                pltpu.make_async_copy(
                    w.at[idx_v.at[p]], gbuf.at[p, comp], gsem[p]).wait()

        pltpu.sync_copy(idx_slice(0), idx_v.at[0])
        fire_gathers(0)
        pltpu.async_copy(idx_slice(1), idx_v.at[1], isem[1])

        @pl.loop(0, n_chunks // 2)
        def _(h):
            for p in range(2):
                g = 2 * h + p

                @pl.when(g + 1 < n_chunks)
                def _():
                    pltpu.make_async_copy(
                        idx_slice(g + 1), idx_v.at[1 - p], isem[1 - p]).wait()
                    fire_gathers(1 - p)

                wait_gathers(p)

                @pl.when(g + 2 < n_chunks)
                def _():
                    pltpu.async_copy(idx_slice(g + 2), idx_v.at[p], isem[p])

                @pl.when(g >= 1)
                def _():
                    pltpu.make_async_copy(obuf, out_slice(g - 1), osem).wait()

                @pl.loop(0, CHUNK)
                def _(t):
                    t_idx = jnp.full((LANES,), t, jnp.int32)
                    for q in range(ROW // LANES):
                        vec = plsc.load_gather(
                            gbuf.at[p], [c_idx, t_idx, d_idx[q]])
                        obuf[t, pl.ds(q * LANES, LANES)] = vec

                pltpu.async_copy(obuf, out_slice(g), osem)

        pltpu.make_async_copy(obuf, out_slice(n_chunks - 1), osem).wait()

    return kern(idx_flat, W_r, W_i, W_j, W_k)


def kernel(input_ids, W_r, W_i, W_j, W_k):
    batch, hist = input_ids.shape
    idx_flat = input_ids.reshape(batch * hist)
    out = _sc_embed(idx_flat, W_r, W_i, W_j, W_k)
    return out.reshape(batch, hist, D, NCOMP)


# speedup vs baseline: 2.2391x; 1.1014x over previous
import functools

import jax
import jax.numpy as jnp
from jax import lax
from jax.experimental import pallas as pl
from jax.experimental.pallas import tpu as pltpu
from jax.experimental.pallas import tpu_sc as plsc

D = 64
NCOMP = 4
ROW = NCOMP * D
LANES = 16

NC = 2
NS = 16
NW = NC * NS

CHUNK = 128


@jax.jit
def _sc_embed(idx_flat, Wri, Wjk):
    n = idx_flat.shape[0]
    per_w = n // NW
    n_chunks = per_w // CHUNK
    assert n_chunks % 2 == 0

    mesh = plsc.VectorSubcoreMesh(core_axis_name="c", subcore_axis_name="s")

    @functools.partial(
        pl.kernel,
        mesh=mesh,
        compiler_params=pltpu.CompilerParams(
            needs_layout_passes=False, use_tc_tiling_on_sc=True),
        out_type=jax.ShapeDtypeStruct((n, ROW), jnp.float32),
        scratch_types=[
            pltpu.VMEM((2, CHUNK), jnp.int32),
            pltpu.VMEM((2, 2, CHUNK, 2 * D), jnp.float32),
            pltpu.VMEM((CHUNK, ROW), jnp.float32),
            pltpu.SemaphoreType.DMA,
            pltpu.SemaphoreType.DMA,
            pltpu.SemaphoreType.DMA,
            pltpu.SemaphoreType.DMA,
            pltpu.SemaphoreType.DMA,
        ],
    )
    def kern(idx_hbm, wri_hbm, wjk_hbm, out_hbm,
             idx_v, gbuf, obuf, gsem0, gsem1, isem0, isem1, osem):
        tables = (wri_hbm, wjk_hbm)
        gsem = (gsem0, gsem1)
        isem = (isem0, isem1)

        wid = lax.axis_index("s") * NC + lax.axis_index("c")
        base = wid * per_w

        lane = lax.broadcasted_iota(jnp.int32, (LANES,), 0)
        pair_idx = (lane >> 1) & 1
        col_idx = [(lane & 1) * D + (lane >> 2) + (q * 4)
                   for q in range(ROW // LANES)]

        def idx_slice(g):
            return idx_hbm.at[pl.ds(base + g * CHUNK, CHUNK)]

        def out_slice(g):
            return out_hbm.at[pl.ds(base + g * CHUNK, CHUNK)]

        def fire_gathers(p):
            for pair, w in enumerate(tables):
                pltpu.async_copy(w.at[idx_v.at[p]], gbuf.at[p, pair], gsem[p])

        def wait_gathers(p):
            for pair, w in enumerate(tables):
                pltpu.make_async_copy(
                    w.at[idx_v.at[p]], gbuf.at[p, pair], gsem[p]).wait()

        pltpu.sync_copy(idx_slice(0), idx_v.at[0])
        fire_gathers(0)
        pltpu.async_copy(idx_slice(1), idx_v.at[1], isem[1])

        @pl.loop(0, n_chunks // 2)
        def _(h):
            for p in range(2):
                g = 2 * h + p

                @pl.when(g + 1 < n_chunks)
                def _():
                    pltpu.make_async_copy(
                        idx_slice(g + 1), idx_v.at[1 - p], isem[1 - p]).wait()
                    fire_gathers(1 - p)

                wait_gathers(p)

                @pl.when(g + 2 < n_chunks)
                def _():
                    pltpu.async_copy(idx_slice(g + 2), idx_v.at[p], isem[p])

                @pl.when(g >= 1)
                def _():
                    pltpu.make_async_copy(obuf, out_slice(g - 1), osem).wait()

                @pl.loop(0, CHUNK)
                def _(t):
                    t_idx = jnp.full((LANES,), t, jnp.int32)
                    for q in range(ROW // LANES):
                        vec = plsc.load_gather(
                            gbuf.at[p], [pair_idx, t_idx, col_idx[q]])
                        obuf[t, pl.ds(q * LANES, LANES)] = vec

                pltpu.async_copy(obuf, out_slice(g), osem)

        pltpu.make_async_copy(obuf, out_slice(n_chunks - 1), osem).wait()

    return kern(idx_flat, Wri, Wjk)


def kernel(input_ids, W_r, W_i, W_j, W_k):
    batch, hist = input_ids.shape
    idx_flat = input_ids.reshape(batch * hist)
    Wri = jnp.concatenate([W_r, W_i], axis=1)
    Wjk = jnp.concatenate([W_j, W_k], axis=1)
    out = _sc_embed(idx_flat, Wri, Wjk)
    return out.reshape(batch, hist, D, NCOMP)


# speedup vs baseline: 6.1387x; 2.7416x over previous
import functools

import jax
import jax.numpy as jnp
from jax import lax
from jax.experimental import pallas as pl
from jax.experimental.pallas import tpu as pltpu
from jax.experimental.pallas import tpu_sc as plsc

D = 64
NCOMP = 4
LANES = 16

NC = 2
NS = 16
NW = NC * NS


@jax.jit
def _sc_embed(idx_t, Wt_r, Wt_i, Wt_j, Wt_k):
    hist, batch = idx_t.shape
    vocab = Wt_r.shape[1]
    d_per_w = D // NW
    n_vec = batch // LANES

    mesh = plsc.VectorSubcoreMesh(core_axis_name="c", subcore_axis_name="s")

    @functools.partial(
        pl.kernel,
        mesh=mesh,
        compiler_params=pltpu.CompilerParams(
            needs_layout_passes=False, use_tc_tiling_on_sc=True),
        out_type=jax.ShapeDtypeStruct((hist, D, NCOMP, batch), jnp.float32),
        scratch_types=[
            pltpu.VMEM((vocab,), jnp.float32),
            pltpu.VMEM((2, batch), jnp.int32),
            pltpu.VMEM((2, batch), jnp.float32),
            pltpu.SemaphoreType.DMA,
            pltpu.SemaphoreType.DMA,
            pltpu.SemaphoreType.DMA,
            pltpu.SemaphoreType.DMA,
        ],
    )
    def kern(idx_hbm, wr_hbm, wi_hbm, wj_hbm, wk_hbm, out_hbm,
             row_v, idxr, obuf, xsem0, xsem1, osem0, osem1):
        xsem = (xsem0, xsem1)
        osem = (osem0, osem1)
        cid = lax.axis_index("c")
        sid = lax.axis_index("s")
        wid = sid * NC + cid

        for dd in range(d_per_w):
            d = wid * d_per_w + dd
            for c, w in enumerate((wr_hbm, wi_hbm, wj_hbm, wk_hbm)):
                pltpu.sync_copy(w.at[d], row_v)
                pltpu.async_copy(idx_hbm.at[0], idxr.at[0], xsem[0])
                pltpu.async_copy(idx_hbm.at[1], idxr.at[1], xsem[1])

                @pl.loop(0, hist // 2)
                def _(hh):
                    for p in range(2):
                        h = 2 * hh + p

                        pltpu.make_async_copy(
                            idx_hbm.at[h], idxr.at[p], xsem[p]).wait()

                        @pl.when(h >= 2)
                        def _():
                            pltpu.make_async_copy(
                                obuf.at[p], out_hbm.at[h - 2, d, c],
                                osem[p]).wait()

                        @pl.loop(0, n_vec)
                        def _(i):
                            iv = idxr[p, pl.ds(i * LANES, LANES)]
                            obuf[p, pl.ds(i * LANES, LANES)] = (
                                plsc.load_gather(row_v, [iv]))

                        pltpu.async_copy(
                            obuf.at[p], out_hbm.at[h, d, c], osem[p])

                        @pl.when(h + 2 < hist)
                        def _():
                            pltpu.async_copy(
                                idx_hbm.at[h + 2], idxr.at[p], xsem[p])

                for hh in (hist - 2, hist - 1):
                    pltpu.make_async_copy(
                        obuf.at[hh % 2], out_hbm.at[hh, d, c],
                        osem[hh % 2]).wait()

    return kern(idx_t, Wt_r, Wt_i, Wt_j, Wt_k)


def kernel(input_ids, W_r, W_i, W_j, W_k):
    batch, hist = input_ids.shape
    idx_t = input_ids.T
    out_t = _sc_embed(idx_t, W_r.T, W_i.T, W_j.T, W_k.T)
    return out_t.transpose(3, 0, 1, 2)


# speedup vs baseline: 7.5416x; 1.2285x over previous
import functools

import jax
import jax.numpy as jnp
from jax import lax
from jax.experimental import pallas as pl
from jax.experimental.pallas import tpu as pltpu
from jax.experimental.pallas import tpu_sc as plsc

D = 64
NCOMP = 4
LANES = 16

NC = 2
NS = 16
NW = NC * NS


@jax.jit
def _sc_embed(idx_t, Wt_r, Wt_i, Wt_j, Wt_k):
    hist, batch = idx_t.shape
    vocab = Wt_r.shape[1]
    d_per_w = D // NW
    n_vec = batch // LANES

    mesh = plsc.VectorSubcoreMesh(core_axis_name="c", subcore_axis_name="s")

    @functools.partial(
        pl.kernel,
        mesh=mesh,
        compiler_params=pltpu.CompilerParams(
            needs_layout_passes=False, use_tc_tiling_on_sc=True),
        out_type=jax.ShapeDtypeStruct((hist, D, NCOMP, batch), jnp.float32),
        scratch_types=[
            pltpu.VMEM((vocab,), jnp.float32),
            pltpu.VMEM((2, batch), jnp.int32),
            pltpu.VMEM((2, batch), jnp.float32),
            pltpu.SemaphoreType.DMA,
            pltpu.SemaphoreType.DMA,
            pltpu.SemaphoreType.DMA,
            pltpu.SemaphoreType.DMA,
        ],
    )
    def kern(idx_hbm, wr_hbm, wi_hbm, wj_hbm, wk_hbm, out_hbm,
             row_v, idxr, obuf, xsem0, xsem1, osem0, osem1):
        xsem = (xsem0, xsem1)
        osem = (osem0, osem1)
        cid = lax.axis_index("c")
        sid = lax.axis_index("s")
        wid = sid * NC + cid

        UNROLL = 8

        for dd in range(d_per_w):
            d = wid * d_per_w + dd
            for c, w in enumerate((wr_hbm, wi_hbm, wj_hbm, wk_hbm)):
                pltpu.sync_copy(w.at[d], row_v)
                pltpu.async_copy(idx_hbm.at[0], idxr.at[0], xsem[0])
                pltpu.async_copy(idx_hbm.at[1], idxr.at[1], xsem[1])

                @pl.loop(0, hist // 2)
                def _(hh):
                    for p in range(2):
                        h = 2 * hh + p

                        pltpu.make_async_copy(
                            idx_hbm.at[h], idxr.at[p], xsem[p]).wait()

                        @pl.when(h >= 2)
                        def _():
                            pltpu.make_async_copy(
                                obuf.at[p], out_hbm.at[h - 2, d, c],
                                osem[p]).wait()

                        @pl.loop(0, n_vec // UNROLL)
                        def _(i):
                            for u in range(UNROLL):
                                off = (i * UNROLL + u) * LANES
                                iv = idxr[p, pl.ds(off, LANES)]
                                obuf[p, pl.ds(off, LANES)] = (
                                    plsc.load_gather(row_v, [iv]))

                        pltpu.async_copy(
                            obuf.at[p], out_hbm.at[h, d, c], osem[p])

                        @pl.when(h + 2 < hist)
                        def _():
                            pltpu.async_copy(
                                idx_hbm.at[h + 2], idxr.at[p], xsem[p])

                for hh in (hist - 2, hist - 1):
                    pltpu.make_async_copy(
                        obuf.at[hh % 2], out_hbm.at[hh, d, c],
                        osem[hh % 2]).wait()

    return kern(idx_t, Wt_r, Wt_i, Wt_j, Wt_k)


def kernel(input_ids, W_r, W_i, W_j, W_k):
    batch, hist = input_ids.shape
    idx_t = input_ids.T
    out_t = _sc_embed(idx_t, W_r.T, W_i.T, W_j.T, W_k.T)
    return out_t.transpose(3, 0, 1, 2)


# speedup vs baseline: 12.3779x; 1.6413x over previous
import functools

import jax
import jax.numpy as jnp
from jax import lax
from jax.experimental import pallas as pl
from jax.experimental.pallas import tpu as pltpu
from jax.experimental.pallas import tpu_sc as plsc

D = 64
NCOMP = 4
LANES = 16

NC = 2
NS = 16
NW = NC * NS


@jax.jit
def _sc_embed(idx_t, Wt_r, Wt_i, Wt_j, Wt_k):
    hist, batch = idx_t.shape
    vocab = Wt_r.shape[1]
    d_per_w = D // NW
    n_vec = batch // LANES

    mesh = plsc.VectorSubcoreMesh(core_axis_name="c", subcore_axis_name="s")

    @functools.partial(
        pl.kernel,
        mesh=mesh,
        compiler_params=pltpu.CompilerParams(
            needs_layout_passes=False, use_tc_tiling_on_sc=True),
        out_type=jax.ShapeDtypeStruct((hist, D, NCOMP, batch), jnp.float32),
        scratch_types=[
            pltpu.VMEM((vocab,), jnp.float32),
            pltpu.VMEM((2, batch), jnp.int32),
            pltpu.VMEM((2, batch), jnp.float32),
            pltpu.SemaphoreType.DMA,
            pltpu.SemaphoreType.DMA,
            pltpu.SemaphoreType.DMA,
            pltpu.SemaphoreType.DMA,
        ],
    )
    def kern(idx_hbm, wr_hbm, wi_hbm, wj_hbm, wk_hbm, out_hbm,
             row_v, idxr, obuf, xsem0, xsem1, osem0, osem1):
        xsem = (xsem0, xsem1)
        osem = (osem0, osem1)
        cid = lax.axis_index("c")
        sid = lax.axis_index("s")
        wid = sid * NC + cid

        UNROLL = 8

        for dd in range(d_per_w):
            d = wid * d_per_w + dd
            for c, w in enumerate((wr_hbm, wi_hbm, wj_hbm, wk_hbm)):
                pltpu.sync_copy(w.at[d], row_v)
                pltpu.async_copy(idx_hbm.at[0], idxr.at[0], xsem[0])
                pltpu.async_copy(idx_hbm.at[1], idxr.at[1], xsem[1])

                @pl.loop(0, hist // 2)
                def _(hh):
                    for p in range(2):
                        h = 2 * hh + p

                        pltpu.make_async_copy(
                            idx_hbm.at[h], idxr.at[p], xsem[p]).wait()

                        @pl.when(h >= 2)
                        def _():
                            pltpu.make_async_copy(
                                obuf.at[p], out_hbm.at[h - 2, d, c],
                                osem[p]).wait()

                        @pl.loop(0, n_vec // UNROLL)
                        def _(i):
                            offs = [(i * UNROLL + u) * LANES
                                    for u in range(UNROLL)]
                            ivs = [idxr[p, pl.ds(off, LANES)]
                                   for off in offs]
                            vals = [plsc.load_gather(row_v, [iv])
                                    for iv in ivs]
                            for off, val in zip(offs, vals):
                                obuf[p, pl.ds(off, LANES)] = val

                        pltpu.async_copy(
                            obuf.at[p], out_hbm.at[h, d, c], osem[p])

                        @pl.when(h + 2 < hist)
                        def _():
                            pltpu.async_copy(
                                idx_hbm.at[h + 2], idxr.at[p], xsem[p])

                for hh in (hist - 2, hist - 1):
                    pltpu.make_async_copy(
                        obuf.at[hh % 2], out_hbm.at[hh, d, c],
                        osem[hh % 2]).wait()

    return kern(idx_t, Wt_r, Wt_i, Wt_j, Wt_k)


def kernel(input_ids, W_r, W_i, W_j, W_k):
    batch, hist = input_ids.shape
    idx_t = input_ids.T
    out_t = _sc_embed(idx_t, W_r.T, W_i.T, W_j.T, W_k.T)
    return out_t.transpose(3, 0, 1, 2)


# speedup vs baseline: 16.3160x; 1.3182x over previous
import functools

import jax
import jax.numpy as jnp
from jax import lax
from jax.experimental import pallas as pl
from jax.experimental.pallas import tpu as pltpu
from jax.experimental.pallas import tpu_sc as plsc

D = 64
NCOMP = 4
LANES = 16

NC = 2
NS = 16
NW = NC * NS


@jax.jit
def _sc_embed(idx_t, Wt_r, Wt_i, Wt_j, Wt_k):
    hist, batch = idx_t.shape
    vocab = Wt_r.shape[1]
    d_per_w = D // NW
    n_vec = batch // LANES

    mesh = plsc.VectorSubcoreMesh(core_axis_name="c", subcore_axis_name="s")

    @functools.partial(
        pl.kernel,
        mesh=mesh,
        compiler_params=pltpu.CompilerParams(
            needs_layout_passes=False, use_tc_tiling_on_sc=True),
        out_type=jax.ShapeDtypeStruct((hist, D, NCOMP, batch), jnp.float32),
        scratch_types=[
            pltpu.VMEM_SHARED((hist * batch,), jnp.int32),
            pltpu.VMEM((vocab,), jnp.float32),
            pltpu.VMEM((2, batch), jnp.int32),
            pltpu.VMEM((2, batch), jnp.float32),
            pltpu.SemaphoreType.DMA,
            pltpu.SemaphoreType.DMA,
            pltpu.SemaphoreType.DMA,
            pltpu.SemaphoreType.DMA,
        ],
    )
    def kern(idx_hbm, wr_hbm, wi_hbm, wj_hbm, wk_hbm, out_hbm,
             sidx, row_v, idxr, obuf, xsem0, xsem1, osem0, osem1):
        xsem = (xsem0, xsem1)
        osem = (osem0, osem1)
        cid = lax.axis_index("c")
        sid = lax.axis_index("s")
        wid = sid * NC + cid

        for r in range((hist + NS - 1) // NS):
            h0 = sid + NS * r

            @pl.when(h0 < hist)
            def _():
                pltpu.sync_copy(idx_hbm.at[h0],
                                sidx.at[pl.ds(h0 * batch, batch)])

        plsc.subcore_barrier()

        UNROLL = 8

        for dd in range(d_per_w):
            d = wid * d_per_w + dd
            for c, w in enumerate((wr_hbm, wi_hbm, wj_hbm, wk_hbm)):
                pltpu.sync_copy(w.at[d], row_v)
                pltpu.async_copy(sidx.at[pl.ds(0, batch)], idxr.at[0], xsem[0])
                pltpu.async_copy(sidx.at[pl.ds(batch, batch)], idxr.at[1], xsem[1])

                @pl.loop(0, hist // 2)
                def _(hh):
                    for p in range(2):
                        h = 2 * hh + p

                        pltpu.make_async_copy(
                            sidx.at[pl.ds(h * batch, batch)], idxr.at[p], xsem[p]).wait()

                        @pl.when(h >= 2)
                        def _():
                            pltpu.make_async_copy(
                                obuf.at[p], out_hbm.at[h - 2, d, c],
                                osem[p]).wait()

                        @pl.loop(0, n_vec // UNROLL)
                        def _(i):
                            offs = [(i * UNROLL + u) * LANES
                                    for u in range(UNROLL)]
                            ivs = [idxr[p, pl.ds(off, LANES)]
                                   for off in offs]
                            vals = [plsc.load_gather(row_v, [iv])
                                    for iv in ivs]
                            for off, val in zip(offs, vals):
                                obuf[p, pl.ds(off, LANES)] = val

                        pltpu.async_copy(
                            obuf.at[p], out_hbm.at[h, d, c], osem[p])

                        @pl.when(h + 2 < hist)
                        def _():
                            pltpu.async_copy(
                                sidx.at[pl.ds((h + 2) * batch, batch)],
                                idxr.at[p], xsem[p])

                for hh in (hist - 2, hist - 1):
                    pltpu.make_async_copy(
                        obuf.at[hh % 2], out_hbm.at[hh, d, c],
                        osem[hh % 2]).wait()

    return kern(idx_t, Wt_r, Wt_i, Wt_j, Wt_k)


def kernel(input_ids, W_r, W_i, W_j, W_k):
    batch, hist = input_ids.shape
    idx_t = input_ids.T
    out_t = _sc_embed(idx_t, W_r.T, W_i.T, W_j.T, W_k.T)
    return out_t.transpose(3, 0, 1, 2)


# speedup vs baseline: 16.5561x; 1.0147x over previous
import functools

import jax
import jax.numpy as jnp
from jax import lax
from jax.experimental import pallas as pl
from jax.experimental.pallas import tpu as pltpu
from jax.experimental.pallas import tpu_sc as plsc

D = 64
NCOMP = 4
LANES = 16

NC = 2
NS = 16
NW = NC * NS


@jax.jit
def _sc_embed(idx_t, Wt_r, Wt_i, Wt_j, Wt_k):
    hist, batch = idx_t.shape
    vocab = Wt_r.shape[1]
    d_per_w = D // NW
    n_vec = batch // LANES

    mesh = plsc.VectorSubcoreMesh(core_axis_name="c", subcore_axis_name="s")

    @functools.partial(
        pl.kernel,
        mesh=mesh,
        compiler_params=pltpu.CompilerParams(
            needs_layout_passes=False, use_tc_tiling_on_sc=True),
        out_type=jax.ShapeDtypeStruct((hist, D, NCOMP, batch), jnp.float32),
        scratch_types=[
            pltpu.VMEM_SHARED((hist * batch,), jnp.int32),
            pltpu.VMEM((vocab,), jnp.float32),
            pltpu.VMEM((2, batch), jnp.int32),
            pltpu.VMEM((2, batch), jnp.float32),
            pltpu.SemaphoreType.DMA,
            pltpu.SemaphoreType.DMA,
            pltpu.SemaphoreType.DMA,
            pltpu.SemaphoreType.DMA,
        ],
    )
    def kern(idx_hbm, wr_hbm, wi_hbm, wj_hbm, wk_hbm, out_hbm,
             sidx, row_v, idxr, obuf, xsem0, xsem1, osem0, osem1):
        xsem = (xsem0, xsem1)
        osem = (osem0, osem1)
        cid = lax.axis_index("c")
        sid = lax.axis_index("s")
        wid = sid * NC + cid

        for r in range((hist + NS - 1) // NS):
            h0 = sid + NS * r

            @pl.when(h0 < hist)
            def _():
                pltpu.sync_copy(idx_hbm.at[h0],
                                sidx.at[pl.ds(h0 * batch, batch)])

        plsc.subcore_barrier()

        UNROLL = 16

        for dd in range(d_per_w):
            d = wid * d_per_w + dd
            for c, w in enumerate((wr_hbm, wi_hbm, wj_hbm, wk_hbm)):
                pltpu.sync_copy(w.at[d], row_v)
                pltpu.async_copy(sidx.at[pl.ds(0, batch)], idxr.at[0], xsem[0])
                pltpu.async_copy(sidx.at[pl.ds(batch, batch)], idxr.at[1], xsem[1])

                @pl.loop(0, hist // 2)
                def _(hh):
                    for p in range(2):
                        h = 2 * hh + p

                        pltpu.make_async_copy(
                            sidx.at[pl.ds(h * batch, batch)], idxr.at[p], xsem[p]).wait()

                        @pl.when(h >= 2)
                        def _():
                            pltpu.make_async_copy(
                                obuf.at[p], out_hbm.at[h - 2, d, c],
                                osem[p]).wait()

                        @pl.loop(0, n_vec // UNROLL)
                        def _(i):
                            offs = [(i * UNROLL + u) * LANES
                                    for u in range(UNROLL)]
                            ivs = [idxr[p, pl.ds(off, LANES)]
                                   for off in offs]
                            vals = [plsc.load_gather(row_v, [iv])
                                    for iv in ivs]
                            for off, val in zip(offs, vals):
                                obuf[p, pl.ds(off, LANES)] = val

                        pltpu.async_copy(
                            obuf.at[p], out_hbm.at[h, d, c], osem[p])

                        @pl.when(h + 2 < hist)
                        def _():
                            pltpu.async_copy(
                                sidx.at[pl.ds((h + 2) * batch, batch)],
                                idxr.at[p], xsem[p])

                for hh in (hist - 2, hist - 1):
                    pltpu.make_async_copy(
                        obuf.at[hh % 2], out_hbm.at[hh, d, c],
                        osem[hh % 2]).wait()

    return kern(idx_t, Wt_r, Wt_i, Wt_j, Wt_k)


def kernel(input_ids, W_r, W_i, W_j, W_k):
    batch, hist = input_ids.shape
    idx_t = input_ids.T
    out_t = _sc_embed(idx_t, W_r.T, W_i.T, W_j.T, W_k.T)
    return out_t.transpose(3, 0, 1, 2)
